# Initial kernel scaffold; baseline (speedup 1.0000x reference)
import jax
import jax.numpy as jnp
from jax.experimental import pallas as pl


def kernel(image, lors):
    raise NotImplementedError("write your pallas kernel here")


# speedup vs baseline: 465.6169x; 465.6169x over previous
import functools

import jax
import jax.numpy as jnp
from jax import lax
from jax.experimental import pallas as pl
from jax.experimental.pallas import tpu as pltpu
from jax.experimental.pallas import tpu_sc as plsc

_GRID = 128
_NC, _NS, _LANES = 2, 16, 16
_NW = _NC * _NS
_NLOR = 65536
_PER_W = _NLOR // _NW
_NSAMP = 128
_NVEC = _NSAMP // _LANES

_XD, _CD = _GRID + 1, 6
_ZWIN, _ZSTEP = 32, 24
_ROWS = _XD * _XD * _CD
_XSTRIDE = _XD * _CD
_YSTRIDE = _CD

_INV_VOX = 0.64
_COORD_OFF = 63.5
_NEG_HALF_INV_SIG2 = -0.02


def _build_table(image):
    p = jnp.pad(image, ((1, 1), (1, 1), (1, 23)))
    parts = []
    for dx in (0, 1):
        for dy in (0, 1):
            sub = p[dx:dx + _XD, dy:dy + _XD, :]
            win = jnp.stack(
                [sub[:, :, _ZSTEP * c:_ZSTEP * c + _ZWIN] for c in range(_CD)],
                axis=2,
            )
            parts.append(win)
    t = jnp.stack(parts, axis=3)
    return t.reshape(_ROWS, 128)


def _rsqrt_nr(x):
    i = lax.bitcast_convert_type(x, jnp.int32)
    i = jnp.int32(0x5F3759DF) - (i >> 1)
    y = lax.bitcast_convert_type(i, jnp.float32)
    for _ in range(3):
        y = y * (1.5 - 0.5 * x * y * y)
    return y


def _floor_i(c):
    t = c.astype(jnp.int32)
    return jnp.where(t.astype(jnp.float32) > c, t - 1, t)


def _sc_project(table, lors_t):
    mesh = plsc.VectorSubcoreMesh(core_axis_name="c", subcore_axis_name="s")

    @functools.partial(
        pl.kernel,
        out_type=jax.ShapeDtypeStruct((_NLOR,), jnp.float32),
        mesh=mesh,
        compiler_params=pltpu.CompilerParams(needs_layout_passes=False),
        scratch_types=[
            pltpu.VMEM((7, _PER_W), jnp.float32),
            pltpu.VMEM((_PER_W * 8 + 128,), jnp.float32),
            pltpu.VMEM((_PER_W,), jnp.float32),
            pltpu.VMEM((_NSAMP,), jnp.int32),
            pltpu.VMEM((_NSAMP,), jnp.int32),
            pltpu.VMEM((_NSAMP, 128), jnp.float32),
            pltpu.VMEM((_NSAMP, 128), jnp.float32),
            pltpu.SemaphoreType.DMA,
            pltpu.SemaphoreType.DMA,
        ],
    )
    def proj(tab_hbm, lors_hbm, out_hbm, lors_v, prep_v, out_v,
             idx0, idx1, rows0, rows1, sem0, sem1):
        wid = lax.axis_index("s") * _NC + lax.axis_index("c")
        base = wid * _PER_W
        pltpu.sync_copy(lors_hbm.at[:, pl.ds(base, _PER_W)], lors_v)

        io32 = lax.iota(jnp.int32, _LANES)
        iof = io32.astype(jnp.float32)
        lane0 = io32 == 0

        def prep_body(g, carry):
            s = pl.ds(g * _LANES, _LANES)
            p1x = lors_v[0, s]
            p1y = lors_v[1, s]
            p1z = lors_v[2, s]
            dx = lors_v[3, s] - p1x
            dy = lors_v[4, s] - p1y
            dz = lors_v[5, s] - p1z
            tof = lors_v[6, s]
            dd = dx * dx + dy * dy + dz * dz + 1e-12
            ell = dd * _rsqrt_nr(dd)
            ibase = (g * _LANES + io32) * 8
            vals = (
                p1x * _INV_VOX + _COORD_OFF,
                p1y * _INV_VOX + _COORD_OFF,
                p1z * _INV_VOX + _COORD_OFF,
                dx * (_INV_VOX / (_NSAMP - 1)),
                dy * (_INV_VOX / (_NSAMP - 1)),
                dz * (_INV_VOX / (_NSAMP - 1)),
                ell * (1.0 / (_NSAMP - 1)),
                0.5 * ell + 2.0 * tof,
            )
            for c, vec in enumerate(vals):
                plsc.store_scatter(prep_v, [ibase + c], vec)
            return carry

        lax.fori_loop(0, _PER_W // _LANES, prep_body, 0)

        def phase_a(i, idxbuf):
            pv = prep_v[pl.ds(i * 8, _LANES)]
            bx = pv[0]
            by = pv[1]
            bz = pv[2]
            ax = pv[3]
            ay = pv[4]
            az = pv[5]
            for v in range(_NVEC):
                kf = iof + float(v * _LANES)
                xs = jnp.clip(_floor_i(bx + kf * ax) + 1, 0, _GRID)
                ys = jnp.clip(_floor_i(by + kf * ay) + 1, 0, _GRID)
                zs = jnp.clip(_floor_i(bz + kf * az) + 1, 0, _GRID)
                cs = (zs * 2731) >> 16
                r = xs * _XSTRIDE + ys * _YSTRIDE + cs
                idxbuf[pl.ds(v * _LANES, _LANES)] = r

        def phase_b(i, rowsbuf):
            pv = prep_v[pl.ds(i * 8, _LANES)]
            bx = pv[0]
            by = pv[1]
            bz = pv[2]
            ax = pv[3]
            ay = pv[4]
            az = pv[5]
            sl = pv[6]
            tc = pv[7]
            acc = jnp.zeros((_LANES,), jnp.float32)
            for v in range(_NVEC):
                kf = iof + float(v * _LANES)
                cx = bx + kf * ax
                cy = by + kf * ay
                cz = bz + kf * az
                fx = _floor_i(cx)
                fy = _floor_i(cy)
                fz = _floor_i(cz)
                frx = cx - fx.astype(jnp.float32)
                fry = cy - fy.astype(jnp.float32)
                frz = cz - fz.astype(jnp.float32)
                wx0 = jnp.where((fx >= 0) & (fx < _GRID), 1.0 - frx, 0.0)
                wx1 = jnp.where((fx >= -1) & (fx < _GRID - 1), frx, 0.0)
                wy0 = jnp.where((fy >= 0) & (fy < _GRID), 1.0 - fry, 0.0)
                wy1 = jnp.where((fy >= -1) & (fy < _GRID - 1), fry, 0.0)
                wz0 = jnp.where((fz >= 0) & (fz < _GRID), 1.0 - frz, 0.0)
                wz1 = jnp.where((fz >= -1) & (fz < _GRID - 1), frz, 0.0)
                zs = jnp.clip(fz + 1, 0, _GRID)
                cs = (zs * 2731) >> 16
                lz = zs - cs * _ZSTEP
                row = io32 + v * _LANES
                v000 = plsc.load_gather(rowsbuf, [row, lz])
                v001 = plsc.load_gather(rowsbuf, [row, lz + 1])
                v010 = plsc.load_gather(rowsbuf, [row, lz + 32])
                v011 = plsc.load_gather(rowsbuf, [row, lz + 33])
                v100 = plsc.load_gather(rowsbuf, [row, lz + 64])
                v101 = plsc.load_gather(rowsbuf, [row, lz + 65])
                v110 = plsc.load_gather(rowsbuf, [row, lz + 96])
                v111 = plsc.load_gather(rowsbuf, [row, lz + 97])
                u00 = wz0 * v000 + wz1 * v001
                u01 = wz0 * v010 + wz1 * v011
                u10 = wz0 * v100 + wz1 * v101
                u11 = wz0 * v110 + wz1 * v111
                val = wx0 * (wy0 * u00 + wy1 * u01) \
                    + wx1 * (wy0 * u10 + wy1 * u11)
                e = kf * sl - tc
                acc = acc + val * jnp.exp(e * e * _NEG_HALF_INV_SIG2)
            total = jnp.sum(acc) * sl
            plsc.store_scatter(out_v, [jnp.full((_LANES,), i, jnp.int32)],
                               jnp.full((_LANES,), total, jnp.float32),
                               mask=lane0)

        phase_a(0, idx0)
        pltpu.async_copy(tab_hbm.at[idx0], rows0, sem0)
        phase_a(1, idx1)
        pltpu.async_copy(tab_hbm.at[idx1], rows1, sem1)

        def main_body(m, carry):
            i = 2 * m
            pltpu.make_async_copy(tab_hbm.at[idx0], rows0, sem0).wait()
            phase_b(i, rows0)
            phase_a(i + 2, idx0)
            pltpu.async_copy(tab_hbm.at[idx0], rows0, sem0)
            pltpu.make_async_copy(tab_hbm.at[idx1], rows1, sem1).wait()
            phase_b(i + 1, rows1)
            phase_a(i + 3, idx1)
            pltpu.async_copy(tab_hbm.at[idx1], rows1, sem1)
            return carry

        lax.fori_loop(0, _PER_W // 2 - 1, main_body, 0)

        pltpu.make_async_copy(tab_hbm.at[idx0], rows0, sem0).wait()
        phase_b(_PER_W - 2, rows0)
        pltpu.make_async_copy(tab_hbm.at[idx1], rows1, sem1).wait()
        phase_b(_PER_W - 1, rows1)

        pltpu.sync_copy(out_v, out_hbm.at[pl.ds(base, _PER_W)])

    return proj(table, lors_t)


def kernel(image, lors):
    table = _build_table(image.astype(jnp.float32))
    lors_t = lors.astype(jnp.float32).T
    return _sc_project(table, lors_t)


# speedup vs baseline: 768.9540x; 1.6515x over previous
import functools

import jax
import jax.numpy as jnp
from jax import lax
from jax.experimental import pallas as pl
from jax.experimental.pallas import tpu as pltpu
from jax.experimental.pallas import tpu_sc as plsc

_GRID = 128
_NC, _NS, _LANES = 2, 16, 16
_NW = _NC * _NS
_NLOR = 65536
_PER_W = _NLOR // _NW
_NSAMP = 128
_NVEC = _NSAMP // _LANES
_NPAIR = _NVEC // 2

_XD, _CD = _GRID + 1, 6
_ZWIN, _ZSTEP = 32, 24
_ROWS = _XD * _XD * _CD
_XSTRIDE = _XD * _CD
_YSTRIDE = _CD

_INV_VOX = 0.64
_COORD_OFF = 63.5
_NEG_HALF_INV_SIG2 = -0.02
_TOF_HW = 25.0
_PSLOT = 16


def _build_table(image):
    p = jnp.pad(image, ((1, 1), (1, 1), (1, 23)))
    parts = []
    for dx in (0, 1):
        for dy in (0, 1):
            sub = p[dx:dx + _XD, dy:dy + _XD, :]
            win = jnp.stack(
                [sub[:, :, _ZSTEP * c:_ZSTEP * c + _ZWIN] for c in range(_CD)],
                axis=2,
            )
            parts.append(win)
    t = jnp.stack(parts, axis=3)
    return t.reshape(_ROWS, 128)


def _rsqrt_nr(x):
    i = lax.bitcast_convert_type(x, jnp.int32)
    i = jnp.int32(0x5F3759DF) - (i >> 1)
    y = lax.bitcast_convert_type(i, jnp.float32)
    for _ in range(3):
        y = y * (1.5 - 0.5 * x * y * y)
    return y


def _floor_i(c):
    t = c.astype(jnp.int32)
    return jnp.where(t.astype(jnp.float32) > c, t - 1, t)


def _inb(i):
    return lax.bitcast_convert_type(i, jnp.uint32) < jnp.uint32(_GRID)


def _axis_window(b, a):
    rs = _rsqrt_nr(a * a)
    ra = a * rs * rs
    t0 = (-1.0 - b) * ra
    t1 = (128.0 - b) * ra
    lo = jnp.minimum(t0, t1)
    hi = jnp.maximum(t0, t1)
    small = jnp.abs(a) < 1e-6
    inside = (b > -1.0) & (b < 128.0)
    lo = jnp.where(small, jnp.where(inside, -1e9, 1e9), lo)
    hi = jnp.where(small, jnp.where(inside, 1e9, -1e9), hi)
    return lo, hi


def _sc_project(table, lors_t):
    mesh = plsc.VectorSubcoreMesh(core_axis_name="c", subcore_axis_name="s")

    @functools.partial(
        pl.kernel,
        out_type=jax.ShapeDtypeStruct((_NLOR,), jnp.float32),
        mesh=mesh,
        compiler_params=pltpu.CompilerParams(needs_layout_passes=False),
        scratch_types=[
            pltpu.VMEM((7, _PER_W), jnp.float32),
            pltpu.VMEM((_PER_W * _PSLOT + 128,), jnp.float32),
            pltpu.VMEM((_PER_W,), jnp.float32),
            pltpu.VMEM((_NSAMP,), jnp.int32),
            pltpu.VMEM((_NSAMP,), jnp.int32),
            pltpu.VMEM((_NSAMP,), jnp.int32),
            pltpu.VMEM((_NSAMP,), jnp.int32),
            pltpu.VMEM((_NSAMP, 128), jnp.float32),
            pltpu.VMEM((_NSAMP, 128), jnp.float32),
            pltpu.VMEM((_NSAMP, 128), jnp.float32),
            pltpu.VMEM((_NSAMP, 128), jnp.float32),
            pltpu.SemaphoreType.DMA,
            pltpu.SemaphoreType.DMA,
            pltpu.SemaphoreType.DMA,
            pltpu.SemaphoreType.DMA,
        ],
    )
    def proj(tab_hbm, lors_hbm, out_hbm, lors_v, prep_v, out_v,
             idx0, idx1, idx2, idx3, rows0, rows1, rows2, rows3,
             sem0, sem1, sem2, sem3):
        wid = lax.axis_index("s") * _NC + lax.axis_index("c")
        base = wid * _PER_W
        pltpu.sync_copy(lors_hbm.at[:, pl.ds(base, _PER_W)], lors_v)

        io32 = lax.iota(jnp.int32, _LANES)
        iof = io32.astype(jnp.float32)
        lane0 = io32 == 0
        splat = tuple(jnp.full((_LANES,), c, jnp.int32) for c in range(12))

        def prep_body(g, carry):
            s = pl.ds(g * _LANES, _LANES)
            p1x = lors_v[0, s]
            p1y = lors_v[1, s]
            p1z = lors_v[2, s]
            dx = lors_v[3, s] - p1x
            dy = lors_v[4, s] - p1y
            dz = lors_v[5, s] - p1z
            tof = lors_v[6, s]
            dd = dx * dx + dy * dy + dz * dz + 1e-12
            rsq = _rsqrt_nr(dd)
            ell = dd * rsq
            bx = p1x * _INV_VOX + _COORD_OFF
            by = p1y * _INV_VOX + _COORD_OFF
            bz = p1z * _INV_VOX + _COORD_OFF
            ax = dx * (_INV_VOX / (_NSAMP - 1))
            ay = dy * (_INV_VOX / (_NSAMP - 1))
            az = dz * (_INV_VOX / (_NSAMP - 1))
            sl = ell * (1.0 / (_NSAMP - 1))
            tc = 0.5 * ell + 2.0 * tof
            rcp_sl = (_NSAMP - 1.0) * rsq
            klo = (tc - _TOF_HW) * rcp_sl
            khi = (tc + _TOF_HW) * rcp_sl
            lox, hix = _axis_window(bx, ax)
            loy, hiy = _axis_window(by, ay)
            loz, hiz = _axis_window(bz, az)
            klo = jnp.maximum(jnp.maximum(klo, lox), jnp.maximum(loy, loz))
            khi = jnp.minimum(jnp.minimum(khi, hix), jnp.minimum(hiy, hiz))
            klo = jnp.clip(klo, 0.0, float(_NSAMP - 1))
            khi = jnp.clip(khi, -1.0, float(_NSAMP - 1))
            klo_i = klo.astype(jnp.int32)
            n = khi.astype(jnp.int32) - klo_i + 2
            npr = jnp.clip((n + 31) >> 5, 0, _NPAIR)
            klo_f = klo_i.astype(jnp.float32)
            ibase = (g * _LANES + io32) * _PSLOT
            vals = (bx + klo_f * ax,
                    by + klo_f * ay,
                    bz + klo_f * az,
                    ax, ay, az, sl,
                    tc - klo_f * sl,
                    float(_NSAMP - 1) - klo_f,
                    npr.astype(jnp.float32))
            for c, vec in enumerate(vals):
                plsc.store_scatter(prep_v, [ibase + c], vec)
            return carry

        lax.fori_loop(0, _PER_W // _LANES, prep_body, 0)

        def load_pv(i):
            pv = prep_v[pl.ds(i * _PSLOT, _LANES)]
            return [pv[splat[c]] for c in range(10)], pv[9].astype(jnp.int32)

        def row_of(cx, cy, cz):
            xs = jnp.clip(_floor_i(cx) + 1, 0, _GRID)
            ys = jnp.clip(_floor_i(cy) + 1, 0, _GRID)
            zs = jnp.clip(_floor_i(cz) + 1, 0, _GRID)
            cs = (zs * 2731) >> 16
            return xs * _XSTRIDE + ys * _YSTRIDE + cs

        def phase_a(i, idxbuf, rowsbuf, sem):
            pvs, npr = load_pv(i)
            bx, by, bz, ax, ay, az = pvs[0], pvs[1], pvs[2], pvs[3], pvs[4], pvs[5]

            def body(p, carry):
                kf = p.astype(jnp.float32) * 32.0 + iof
                kg = kf + 16.0
                idxbuf[pl.ds(p * 32, _LANES)] = \
                    row_of(bx + kf * ax, by + kf * ay, bz + kf * az)
                idxbuf[pl.ds(p * 32 + 16, _LANES)] = \
                    row_of(bx + kg * ax, by + kg * ay, bz + kg * az)
                return carry

            lax.fori_loop(0, npr, body, 0)
            for b in range(1, _NPAIR + 1):
                @pl.when(npr == b)
                def _():
                    pltpu.async_copy(
                        tab_hbm.at[idxbuf.at[pl.ds(0, b * 32)]],
                        rowsbuf.at[pl.ds(0, b * 32)], sem)

        def phase_b(i, idxbuf, rowsbuf, sem):
            pvs, npr = load_pv(i)
            bx, by, bz, ax, ay, az = pvs[0], pvs[1], pvs[2], pvs[3], pvs[4], pvs[5]
            sl, tc, kmax = pvs[6], pvs[7], pvs[8]
            for b in range(1, _NPAIR + 1):
                @pl.when(npr == b)
                def _():
                    pltpu.make_async_copy(
                        tab_hbm.at[idxbuf.at[pl.ds(0, b * 32)]],
                        rowsbuf.at[pl.ds(0, b * 32)], sem).wait()

            def sample_vec(kf, rowbase):
                cx = bx + kf * ax
                cy = by + kf * ay
                cz = bz + kf * az
                fx = _floor_i(cx)
                fy = _floor_i(cy)
                fz = _floor_i(cz)
                frx = cx - fx.astype(jnp.float32)
                fry = cy - fy.astype(jnp.float32)
                frz = cz - fz.astype(jnp.float32)
                gx = fx + 1
                gy = fy + 1
                gz = fz + 1
                wx0 = jnp.where(_inb(fx), 1.0 - frx, 0.0)
                wx1 = jnp.where(_inb(gx), frx, 0.0)
                wy0 = jnp.where(_inb(fy), 1.0 - fry, 0.0)
                wy1 = jnp.where(_inb(gy), fry, 0.0)
                wz0 = jnp.where(_inb(fz), 1.0 - frz, 0.0)
                wz1 = jnp.where(_inb(gz), frz, 0.0)
                zs = jnp.clip(gz, 0, _GRID)
                cs = (zs * 2731) >> 16
                lz = zs - cs * _ZSTEP
                row = rowbase + io32
                v000 = plsc.load_gather(rowsbuf, [row, lz])
                v001 = plsc.load_gather(rowsbuf, [row, lz + 1])
                v010 = plsc.load_gather(rowsbuf, [row, lz + 32])
                v011 = plsc.load_gather(rowsbuf, [row, lz + 33])
                v100 = plsc.load_gather(rowsbuf, [row, lz + 64])
                v101 = plsc.load_gather(rowsbuf, [row, lz + 65])
                v110 = plsc.load_gather(rowsbuf, [row, lz + 96])
                v111 = plsc.load_gather(rowsbuf, [row, lz + 97])
                u00 = wz0 * v000 + wz1 * v001
                u01 = wz0 * v010 + wz1 * v011
                u10 = wz0 * v100 + wz1 * v101
                u11 = wz0 * v110 + wz1 * v111
                val = wx0 * (wy0 * u00 + wy1 * u01) \
                    + wx1 * (wy0 * u10 + wy1 * u11)
                e = kf * sl - tc
                w = jnp.exp(e * e * _NEG_HALF_INV_SIG2)
                w = jnp.where(kf <= kmax, w, 0.0)
                return val * w

            def body(p, acc):
                kf = p.astype(jnp.float32) * 32.0 + iof
                a0 = sample_vec(kf, p * 32)
                a1 = sample_vec(kf + 16.0, p * 32 + 16)
                return acc + a0 + a1

            acc = lax.fori_loop(0, npr, body, jnp.zeros((_LANES,), jnp.float32))
            total = jnp.sum(acc) * pv_scalar(sl)
            plsc.store_scatter(out_v, [jnp.full((_LANES,), i, jnp.int32)],
                               jnp.full((_LANES,), total, jnp.float32),
                               mask=lane0)

        def pv_scalar(vec):
            return vec[0]

        bufs = ((idx0, rows0, sem0), (idx1, rows1, sem1),
                (idx2, rows2, sem2), (idx3, rows3, sem3))
        for j in range(4):
            phase_a(j, *bufs[j])

        def main_body(m, carry):
            i = 4 * m
            for j in range(4):
                phase_b(i + j, *bufs[j])
                phase_a(i + j + 4, *bufs[j])
            return carry

        lax.fori_loop(0, _PER_W // 4 - 1, main_body, 0)

        for j in range(4):
            phase_b(_PER_W - 4 + j, *bufs[j])

        pltpu.sync_copy(out_v, out_hbm.at[pl.ds(base, _PER_W)])

    return proj(table, lors_t)


def kernel(image, lors):
    table = _build_table(image.astype(jnp.float32))
    lors_t = lors.astype(jnp.float32).T
    return _sc_project(table, lors_t)
